# baseline (device time: 27343 ns/iter reference)
import jax
import jax.numpy as jnp
from jax import lax
from jax.experimental import pallas as pl
from jax.experimental.pallas import tpu as pltpu


def kernel(Q, K, V):
    b, seq_per, h, d = Q.shape
    hd = h * d
    scale = d ** -0.5

    Qr = Q.reshape(b, seq_per, hd)
    Kr = K.reshape(b, seq_per, hd)
    Vr = V.reshape(b, seq_per, hd)

    def body(q_ref, k_ref, v_ref, out_ref, kcomm, vcomm, send_sems, recv_sems):
        my_x = lax.axis_index("x")
        my_y = lax.axis_index("y")
        my_z = lax.axis_index("z")
        peer = (1 - my_x, my_y, my_z)

        barrier_sem = pltpu.get_barrier_semaphore()
        pl.semaphore_signal(
            barrier_sem, inc=1, device_id=peer,
            device_id_type=pl.DeviceIdType.MESH,
        )
        pl.semaphore_wait(barrier_sem, 1)

        kcomm[0] = k_ref[...].astype(jnp.bfloat16)
        vcomm[0] = v_ref[...].astype(jnp.bfloat16)

        rdma_k = pltpu.make_async_remote_copy(
            src_ref=kcomm.at[0], dst_ref=kcomm.at[1],
            send_sem=send_sems.at[0], recv_sem=recv_sems.at[0],
            device_id=peer, device_id_type=pl.DeviceIdType.MESH,
        )
        rdma_v = pltpu.make_async_remote_copy(
            src_ref=vcomm.at[0], dst_ref=vcomm.at[1],
            send_sem=send_sems.at[1], recv_sem=recv_sems.at[1],
            device_id=peer, device_id_type=pl.DeviceIdType.MESH,
        )
        rdma_k.start()
        rdma_v.start()
        rdma_k.wait()
        rdma_v.wait()

        dn_qkt = (((1,), (1,)), ((), ()))
        dn_pv = (((1,), (0,)), ((), ()))
        for bb in range(b):
            for hh in range(h):
                c0, c1 = hh * d, (hh + 1) * d
                qh = q_ref[bb, :, c0:c1].astype(jnp.bfloat16)
                s0 = lax.dot_general(
                    qh, kcomm[0, bb, :, c0:c1], dn_qkt,
                    preferred_element_type=jnp.float32,
                )
                s1 = lax.dot_general(
                    qh, kcomm[1, bb, :, c0:c1], dn_qkt,
                    preferred_element_type=jnp.float32,
                )
                s = jnp.concatenate([s0, s1], axis=1) * scale
                m = jnp.max(s, axis=1, keepdims=True)
                p = jnp.exp(s - m)
                denom = jnp.sum(p, axis=1, keepdims=True)
                pb = (p / denom).astype(jnp.bfloat16)
                o = lax.dot_general(
                    pb[:, :seq_per], vcomm[0, bb, :, c0:c1], dn_pv,
                    preferred_element_type=jnp.float32,
                )
                o = o + lax.dot_general(
                    pb[:, seq_per:], vcomm[1, bb, :, c0:c1], dn_pv,
                    preferred_element_type=jnp.float32,
                )
                out_ref[bb, :, c0:c1] = o

    out = pl.pallas_call(
        body,
        out_shape=jax.ShapeDtypeStruct((b, seq_per, hd), jnp.float32),
        in_specs=[pl.BlockSpec(memory_space=pltpu.VMEM)] * 3,
        out_specs=pl.BlockSpec(memory_space=pltpu.VMEM),
        scratch_shapes=[
            pltpu.VMEM((2, b, seq_per, hd), jnp.bfloat16),
            pltpu.VMEM((2, b, seq_per, hd), jnp.bfloat16),
            pltpu.SemaphoreType.DMA((2,)),
            pltpu.SemaphoreType.DMA((2,)),
        ],
        compiler_params=pltpu.CompilerParams(collective_id=0),
    )(Qr, Kr, Vr)
    return out.reshape(b, seq_per, h, d)


# device time: 20129 ns/iter; 1.3584x vs baseline; 1.3584x over previous
import jax
import jax.numpy as jnp
from jax import lax
from jax.experimental import pallas as pl
from jax.experimental.pallas import tpu as pltpu

_DN_QKT = (((1,), (1,)), ((), ()))
_DN_PV = (((1,), (0,)), ((), ()))


def kernel(Q, K, V):
    b, seq_per, h, d = Q.shape
    hd = h * d
    scale = d ** -0.5

    Qr = Q.reshape(b, seq_per, hd)
    Kr = K.reshape(b, seq_per, hd)
    Vr = V.reshape(b, seq_per, hd)

    def body(q_ref, k_ref, v_ref, out_ref, kcomm, vcomm, send_sems, recv_sems):
        my_x = lax.axis_index("x")
        my_y = lax.axis_index("y")
        my_z = lax.axis_index("z")
        peer = (1 - my_x, my_y, my_z)

        barrier_sem = pltpu.get_barrier_semaphore()
        pl.semaphore_signal(
            barrier_sem, inc=1, device_id=peer,
            device_id_type=pl.DeviceIdType.MESH,
        )
        pl.semaphore_wait(barrier_sem, 1)

        kcomm[0] = k_ref[...].astype(jnp.bfloat16)
        vcomm[0] = v_ref[...].astype(jnp.bfloat16)

        rdmas = []
        for i, (comm, bb) in enumerate(
            [(kcomm, 0), (kcomm, 1), (vcomm, 0), (vcomm, 1)]
        ):
            rdmas.append(
                pltpu.make_async_remote_copy(
                    src_ref=comm.at[0, bb], dst_ref=comm.at[1, bb],
                    send_sem=send_sems.at[i], recv_sem=recv_sems.at[i],
                    device_id=peer, device_id_type=pl.DeviceIdType.MESH,
                )
            )
        for r in rdmas:
            r.start()
        rdma_k = rdmas[:2]
        rdma_v = rdmas[2:]

        qh_c = {}
        part = {}
        for bb in range(b):
            for hh in range(h):
                c0, c1 = hh * d, (hh + 1) * d
                qh = (q_ref[bb, :, c0:c1] * scale).astype(jnp.bfloat16)
                qh_c[bb, hh] = qh
                s0 = lax.dot_general(
                    qh, kcomm[0, bb, :, c0:c1], _DN_QKT,
                    preferred_element_type=jnp.float32,
                )
                m0 = jnp.max(s0, axis=1, keepdims=True)
                p0 = jnp.exp(s0 - m0)
                l0 = jnp.sum(p0, axis=1, keepdims=True)
                o0 = lax.dot_general(
                    p0.astype(jnp.bfloat16), vcomm[0, bb, :, c0:c1], _DN_PV,
                    preferred_element_type=jnp.float32,
                )
                part[bb, hh] = (m0, l0, o0)

        for bb in range(b):
            rdma_k[bb].wait_recv()
            merged = {}
            for hh in range(h):
                c0, c1 = hh * d, (hh + 1) * d
                m0, l0, o0 = part[bb, hh]
                s1 = lax.dot_general(
                    qh_c[bb, hh], kcomm[1, bb, :, c0:c1], _DN_QKT,
                    preferred_element_type=jnp.float32,
                )
                m1 = jnp.max(s1, axis=1, keepdims=True)
                m = jnp.maximum(m0, m1)
                p1 = jnp.exp(s1 - m)
                l1 = jnp.sum(p1, axis=1, keepdims=True)
                alpha = jnp.exp(m0 - m)
                l = l0 * alpha + l1
                oacc = o0 * alpha
                merged[hh] = (p1.astype(jnp.bfloat16), l, oacc)
            rdma_v[bb].wait_recv()
            for hh in range(h):
                c0, c1 = hh * d, (hh + 1) * d
                p1, l, oacc = merged[hh]
                o = oacc + lax.dot_general(
                    p1, vcomm[1, bb, :, c0:c1], _DN_PV,
                    preferred_element_type=jnp.float32,
                )
                out_ref[bb, :, c0:c1] = o / l

        for r in rdmas:
            r.wait_send()

    out = pl.pallas_call(
        body,
        out_shape=jax.ShapeDtypeStruct((b, seq_per, hd), jnp.float32),
        in_specs=[pl.BlockSpec(memory_space=pltpu.VMEM)] * 3,
        out_specs=pl.BlockSpec(memory_space=pltpu.VMEM),
        scratch_shapes=[
            pltpu.VMEM((2, b, seq_per, hd), jnp.bfloat16),
            pltpu.VMEM((2, b, seq_per, hd), jnp.bfloat16),
            pltpu.SemaphoreType.DMA((4,)),
            pltpu.SemaphoreType.DMA((4,)),
        ],
        compiler_params=pltpu.CompilerParams(collective_id=0),
    )(Qr, Kr, Vr)
    return out.reshape(b, seq_per, h, d)
